# baseline (speedup 1.0000x reference)
import jax
import jax.numpy as jnp
from jax.experimental import pallas as pl
from jax.experimental.pallas import tpu as pltpu

VOCAB = 1_000_000
EMBED = 16
WINDOW = 2
BLKC = 204_800
CH = 4_096


def _body(ctx_ref, embt0_ref, embt1_ref, wt_ref, b_ref, out_ref):
    l0 = ctx_ref[0] % 128
    l1 = ctx_ref[1] % 128
    lane = jax.lax.broadcasted_iota(jnp.int32, (1, 128), 1)
    x0 = jnp.sum(jnp.where(lane == l0, embt0_ref[...], 0.0), axis=1,
                 keepdims=True)
    x1 = jnp.sum(jnp.where(lane == l1, embt1_ref[...], 0.0), axis=1,
                 keepdims=True)
    x = jnp.concatenate([x0, x1], axis=0)

    def step(k, carry):
        sl = pl.ds(k * CH, CH)
        out_ref[:, sl] = (jnp.sum(wt_ref[:, sl] * x, axis=0, keepdims=True)
                          + b_ref[sl][None, :])
        return carry

    jax.lax.fori_loop(0, BLKC // CH, step, 0)


def kernel(context, emb_table, W, b):
    wt = W.T
    embt = emb_table.T
    grid = (pl.cdiv(VOCAB, BLKC),)
    grid_spec = pltpu.PrefetchScalarGridSpec(
        num_scalar_prefetch=1,
        grid=grid,
        in_specs=[
            pl.BlockSpec((EMBED, 128), lambda i, ctx: (0, ctx[0] // 128)),
            pl.BlockSpec((EMBED, 128), lambda i, ctx: (0, ctx[1] // 128)),
            pl.BlockSpec((EMBED * WINDOW, BLKC), lambda i, ctx: (0, i)),
            pl.BlockSpec((BLKC,), lambda i, ctx: (i,)),
        ],
        out_specs=pl.BlockSpec((1, BLKC), lambda i, ctx: (0, i)),
    )
    out = pl.pallas_call(
        _body,
        grid_spec=grid_spec,
        out_shape=jax.ShapeDtypeStruct((1, VOCAB), jnp.float32),
    )(context, embt, embt, wt, b)
    return out


# speedup vs baseline: 1.1102x; 1.1102x over previous
import jax
import jax.numpy as jnp
from jax.experimental import pallas as pl
from jax.experimental.pallas import tpu as pltpu

VOCAB = 1_000_000
EMBED = 16
WINDOW = 2
BLKC = 131_072


def _body(ctx_ref, embt0_ref, embt1_ref, wt_ref, b_ref, out_ref):
    l0 = ctx_ref[0] % 128
    l1 = ctx_ref[1] % 128
    lane = jax.lax.broadcasted_iota(jnp.int32, (1, 128), 1)
    x0 = jnp.sum(jnp.where(lane == l0, embt0_ref[...], 0.0), axis=1,
                 keepdims=True)
    x1 = jnp.sum(jnp.where(lane == l1, embt1_ref[...], 0.0), axis=1,
                 keepdims=True)
    x = jnp.concatenate([x0, x1], axis=0)
    out_ref[...] = (jnp.sum(wt_ref[...] * x, axis=0, keepdims=True)
                    + b_ref[...][None, :])


def kernel(context, emb_table, W, b):
    wt = W.T
    embt = emb_table.T
    grid = (pl.cdiv(VOCAB, BLKC),)
    grid_spec = pltpu.PrefetchScalarGridSpec(
        num_scalar_prefetch=1,
        grid=grid,
        in_specs=[
            pl.BlockSpec((EMBED, 128), lambda i, ctx: (0, ctx[0] // 128)),
            pl.BlockSpec((EMBED, 128), lambda i, ctx: (0, ctx[1] // 128)),
            pl.BlockSpec((EMBED * WINDOW, BLKC), lambda i, ctx: (0, i)),
            pl.BlockSpec((BLKC,), lambda i, ctx: (i,)),
        ],
        out_specs=pl.BlockSpec((1, BLKC), lambda i, ctx: (0, i)),
    )
    out = pl.pallas_call(
        _body,
        grid_spec=grid_spec,
        out_shape=jax.ShapeDtypeStruct((1, VOCAB), jnp.float32),
    )(context, embt, embt, wt, b)
    return out
